# Initial kernel scaffold; baseline (speedup 1.0000x reference)
import jax
import jax.numpy as jnp
from jax.experimental import pallas as pl


def kernel(x, edge_index, edge_attr, batch, edge_batch, W1, b1, W2, b2):
    raise NotImplementedError("write your pallas kernel here")


# speedup vs baseline: 6.4292x; 6.4292x over previous
import functools

import jax
import jax.numpy as jnp
from jax import lax
from jax.experimental import pallas as pl
from jax.experimental.pallas import tpu as pltpu
from jax.experimental.pallas import tpu_sc as plsc

N = 10000
E = 320000
D = 128
DE = 16
G = 128
H = 256

_INFO = plsc.get_sparse_core_info()
NC = _INFO.num_cores
NS = _INFO.num_subcores
NW = NC * NS

CX = 80
NCX = N // CX
CE = 2000
NCE = E // CE
NEJ = NCE // NW


def _sc_pool_body(x_hbm, b_hbm, ea_hbm, eb_hbm,
                  xp_hbm, ep_hbm, cp_hbm,
                  xbuf, bbuf, abuf, ebbuf, acc_x, acc_e, acc_c,
                  sem0, sem1):
    wid = lax.axis_index("s") * NC + lax.axis_index("c")

    zero16 = jnp.zeros((16,), jnp.float32)
    ones16 = jnp.full((16,), 1.0, jnp.float32)
    sems = [sem0, sem1]

    def _zx(i, _):
        acc_x[pl.ds(i * 16, 16)] = zero16
        return 0
    lax.fori_loop(0, G * D // 16, _zx, 0)

    def _ze(i, _):
        acc_e[pl.ds(i * 16, 16)] = zero16
        acc_c[pl.ds(i * 16, 16)] = zero16
        return 0
    lax.fori_loop(0, G * DE // 16, _ze, 0)

    def _estart(j, slot):
        base = (wid + j * NW) * CE
        h1 = pltpu.make_async_copy(ea_hbm.at[pl.ds(base * DE, CE * DE)],
                                   abuf.at[slot], sems[slot])
        h2 = pltpu.make_async_copy(eb_hbm.at[pl.ds(base, CE)],
                                   ebbuf.at[slot], sems[slot])
        h1.start()
        h2.start()
        return h1, h2

    def _eprocess(slot):
        ab = abuf.at[slot]
        eb = ebbuf.at[slot]
        segv0 = eb[pl.ds(0, 16)]
        init = (segv0[0], jnp.float32(0.0),
                zero16, zero16, zero16, zero16)

        def _group(g, carry):
            cur, cnt, a0, a1, a2, a3 = carry
            e0 = g * 16
            segv = eb[pl.ds(e0, 16)]
            s0 = segv[0]
            s15 = segv[15]
            fast = jnp.logical_and(s0 == cur, s15 == cur)

            rows = [ab[pl.ds((e0 + l) * DE, 16)] for l in range(16)]
            g0 = (rows[0] + rows[4]) + (rows[8] + rows[12])
            g1 = (rows[1] + rows[5]) + (rows[9] + rows[13])
            g2 = (rows[2] + rows[6]) + (rows[10] + rows[14])
            g3 = (rows[3] + rows[7]) + (rows[11] + rows[15])

            @pl.when(jnp.logical_not(fast))
            def _slow():
                plsc.addupdate(acc_e.at[pl.ds(cur * DE, 16)],
                               (a0 + a1) + (a2 + a3))
                plsc.addupdate(acc_c.at[pl.ds(cur * DE, 16)],
                               lax.broadcast_in_dim(cnt, (16,), ()))
                for l in range(16):
                    seg = segv[l]
                    plsc.addupdate(acc_e.at[pl.ds(seg * DE, 16)], rows[l])
                    plsc.addupdate(acc_c.at[pl.ds(seg * DE, 16)], ones16)

            m = jnp.where(fast, 1.0, 0.0)
            mv = lax.broadcast_in_dim(m, (16,), ())
            return (jnp.where(fast, cur, s15),
                    jnp.where(fast, cnt + 16.0, 0.0),
                    (a0 + g0) * mv,
                    (a1 + g1) * mv,
                    (a2 + g2) * mv,
                    (a3 + g3) * mv)

        cur, cnt, a0, a1, a2, a3 = lax.fori_loop(0, CE // 16, _group, init)
        plsc.addupdate(acc_e.at[pl.ds(cur * DE, 16)], (a0 + a1) + (a2 + a3))
        plsc.addupdate(acc_c.at[pl.ds(cur * DE, 16)],
                       lax.broadcast_in_dim(cnt, (16,), ()))

    handles = _estart(0, 0)
    for j in range(NEJ):
        nxt = _estart(j + 1, (j + 1) % 2) if j + 1 < NEJ else None
        for h in handles:
            h.wait()
        _eprocess(j % 2)
        handles = nxt

    for j in range((NCX + NW - 1) // NW):
        c = wid + j * NW

        @pl.when(c < NCX)
        def _():
            base = c * CX
            pltpu.sync_copy(x_hbm.at[pl.ds(base * D, CX * D)], xbuf)
            pltpu.sync_copy(b_hbm.at[pl.ds(base, CX)], bbuf)

            def _xgroup(g, _):
                i0 = g * 16
                segv = bbuf[pl.ds(i0, 16)]
                for l in range(16):
                    seg = segv[l]
                    row0 = (i0 + l) * D
                    dst0 = seg * D
                    for k in range(D // 16):
                        plsc.addupdate(acc_x.at[pl.ds(dst0 + k * 16, 16)],
                                       xbuf[pl.ds(row0 + k * 16, 16)])
                return 0
            lax.fori_loop(0, CX // 16, _xgroup, 0)

    pltpu.sync_copy(acc_x, xp_hbm.at[wid])
    pltpu.sync_copy(acc_e, ep_hbm.at[wid])
    pltpu.sync_copy(acc_c, cp_hbm.at[wid])


_sc_pool_inner = functools.partial(
    pl.kernel,
    out_type=(
        jax.ShapeDtypeStruct((NW, G * D), jnp.float32),
        jax.ShapeDtypeStruct((NW, G * DE), jnp.float32),
        jax.ShapeDtypeStruct((NW, G * DE), jnp.float32),
    ),
    mesh=plsc.VectorSubcoreMesh(core_axis_name="c", subcore_axis_name="s"),
    compiler_params=pltpu.CompilerParams(use_tc_tiling_on_sc=False),
    scratch_types=[
        pltpu.VMEM((CX * D,), jnp.float32),
        pltpu.VMEM((CX,), jnp.int32),
        pltpu.VMEM((2, CE * DE), jnp.float32),
        pltpu.VMEM((2, CE), jnp.int32),
        pltpu.VMEM((G * D,), jnp.float32),
        pltpu.VMEM((G * DE,), jnp.float32),
        pltpu.VMEM((G * DE,), jnp.float32),
        pltpu.SemaphoreType.DMA,
        pltpu.SemaphoreType.DMA,
    ],
)(_sc_pool_body)


def _sc_pool(x, batch, edge_attr, edge_batch):
    xp, ep, cp = _sc_pool_inner(x.reshape(N * D), batch,
                                edge_attr.reshape(E * DE), edge_batch)
    return (xp.reshape(NW, G, D), ep.reshape(NW, G, DE),
            cp.reshape(NW, G, DE))


def _mlp_body(xp_ref, ep_ref, cp_ref, w1a_ref, w1b_ref, w1c_ref,
              b1_ref, w2_ref, b2_ref, o_ref):
    xs = jnp.sum(xp_ref[...], axis=0)
    es = jnp.sum(ep_ref[...], axis=0)
    cs = jnp.sum(cp_ref[...], axis=0)
    cnt = jnp.maximum(cs[:, 0:1], 1.0)
    h = (jnp.dot(xs * 0.1, w1a_ref[...], preferred_element_type=jnp.float32)
         + jnp.dot(es * 0.05, w1b_ref[...], preferred_element_type=jnp.float32)
         + jnp.dot(es / cnt, w1c_ref[...], preferred_element_type=jnp.float32)
         + b1_ref[...])
    h = jnp.where(h > 0, h, 0.05 * h)
    o = jnp.dot(h, w2_ref[...], preferred_element_type=jnp.float32)
    o_ref[...] = (o + b2_ref[0, 0]) * 0.25


def _mlp(xp, ep, cp, w1a, w1b, w1c, b1, w2p, b2):
    return pl.pallas_call(
        _mlp_body,
        out_shape=jax.ShapeDtypeStruct((G, 128), jnp.float32),
    )(xp, ep, cp, w1a, w1b, w1c, b1, w2p, b2)


def kernel(x, edge_index, edge_attr, batch, edge_batch, W1, b1, W2, b2):
    del edge_index
    batch = batch.astype(jnp.int32)
    edge_batch = edge_batch.astype(jnp.int32)
    xp, ep, cp = _sc_pool(x, batch, edge_attr, edge_batch)
    w1a = W1[:D]
    w1b = W1[D:D + DE]
    w1c = W1[D + DE:]
    w2p = jnp.pad(W2, ((0, 0), (0, 127)))
    o = _mlp(xp, ep, cp, w1a, w1b, w1c,
             b1.reshape(1, H), w2p, b2.reshape(1, 1))
    return o[:, 0:1]


# speedup vs baseline: 8.2674x; 1.2859x over previous
import functools

import jax
import jax.numpy as jnp
from jax import lax
from jax.experimental import pallas as pl
from jax.experimental.pallas import tpu as pltpu
from jax.experimental.pallas import tpu_sc as plsc

N = 10000
E = 320000
D = 128
DE = 16
G = 128
H = 256

_INFO = plsc.get_sparse_core_info()
NC = _INFO.num_cores
NS = _INFO.num_subcores
NW = NC * NS

CX = 80
NCX = N // CX
EC = 1280
NEC = E // EC
NEJ = (NEC + NW - 1) // NW


def _sc_pool_body(x_hbm, b_hbm, ea_hbm, eb_hbm,
                  xp_hbm, ep_hbm, cp_hbm,
                  xbuf, bbuf, abuf, ebbuf, acc_x, acc_e, acc_c,
                  sem0, sem1):
    wid = lax.axis_index("s") * NC + lax.axis_index("c")

    zero16 = jnp.zeros((16,), jnp.float32)
    ones16 = jnp.full((16,), 1.0, jnp.float32)
    sems = [sem0, sem1]

    def _zx(i, _):
        acc_x[pl.ds(i * 16, 16)] = zero16
        return 0
    lax.fori_loop(0, G * D // 16, _zx, 0)

    def _ze(i, _):
        acc_e[pl.ds(i * 16, 16)] = zero16
        return 0
    lax.fori_loop(0, G * DE * 16 // 16, _ze, 0)

    def _zc(i, _):
        acc_c[pl.ds(i * 16, 16)] = zero16
        return 0
    lax.fori_loop(0, G * 16 // 16, _zc, 0)

    def _estart(j, slot):
        c = wid + j * NW
        base = c * EC
        h1 = pltpu.make_async_copy(ea_hbm.at[:, pl.ds(base, EC)],
                                   abuf.at[slot], sems[slot])
        h2 = pltpu.make_async_copy(eb_hbm.at[pl.ds(base, EC)],
                                   ebbuf.at[slot], sems[slot])
        h1.start()
        h2.start()
        return h1, h2

    def _eprocess(slot):
        ab = abuf.at[slot]
        eb = ebbuf.at[slot]
        segv0 = eb[pl.ds(0, 16)]
        init = (segv0[0], jnp.float32(0.0)) + (zero16,) * 16

        def _group(g, carry):
            cur = carry[0]
            cnt = carry[1]
            accs = carry[2:]
            e0 = g * 16
            segv = eb[pl.ds(e0, 16)]
            s0 = segv[0]
            s15 = segv[15]
            fast = jnp.logical_and(s0 == cur, s15 == cur)

            rows = [ab[f, pl.ds(e0, 16)] for f in range(DE)]

            @pl.when(jnp.logical_not(fast))
            def _slow():
                for f in range(DE):
                    plsc.addupdate(acc_e.at[pl.ds((cur * DE + f) * 16, 16)],
                                   accs[f])
                plsc.addupdate(acc_c.at[pl.ds(cur * 16, 16)],
                               lax.broadcast_in_dim(cnt * 0.0625, (16,), ()))
                def _piece(l):
                    sl = segv[l]
                    slv = lax.broadcast_in_dim(sl, (16,), ())
                    mk = jnp.where(segv == slv, ones16, zero16)
                    for f in range(DE):
                        plsc.addupdate(
                            acc_e.at[pl.ds((sl * DE + f) * 16, 16)],
                            rows[f] * mk)
                    plsc.addupdate(acc_c.at[pl.ds(sl * 16, 16)], mk)
                _piece(0)
                for l in range(1, 16):
                    @pl.when(segv[l] != segv[l - 1])
                    def _():
                        _piece(l)

            m = jnp.where(fast, 1.0, 0.0)
            mv = lax.broadcast_in_dim(m, (16,), ())
            new_accs = tuple((accs[f] + rows[f]) * mv for f in range(DE))
            return (jnp.where(fast, cur, s15),
                    jnp.where(fast, cnt + 16.0, 0.0)) + new_accs

        out = lax.fori_loop(0, EC // 16, _group, init)
        cur = out[0]
        cnt = out[1]
        accs = out[2:]
        for f in range(DE):
            plsc.addupdate(acc_e.at[pl.ds((cur * DE + f) * 16, 16)], accs[f])
        plsc.addupdate(acc_c.at[pl.ds(cur * 16, 16)],
                       lax.broadcast_in_dim(cnt * 0.0625, (16,), ()))

    def _ewait(slot):
        pltpu.make_async_copy(ea_hbm.at[:, pl.ds(0, EC)], abuf.at[slot],
                              sems[slot]).wait()
        pltpu.make_async_copy(eb_hbm.at[pl.ds(0, EC)], ebbuf.at[slot],
                              sems[slot]).wait()

    _estart(0, 0)
    for j in range(NEJ):
        if j + 1 < NEJ:
            @pl.when(wid + (j + 1) * NW < NEC)
            def _():
                _estart(j + 1, (j + 1) % 2)

        @pl.when(wid + j * NW < NEC)
        def _():
            _ewait(j % 2)
            _eprocess(j % 2)

    for j in range((NCX + NW - 1) // NW):
        c = wid + j * NW

        @pl.when(c < NCX)
        def _():
            base = c * CX
            pltpu.sync_copy(x_hbm.at[pl.ds(base * D, CX * D)], xbuf)
            pltpu.sync_copy(b_hbm.at[pl.ds(base, CX)], bbuf)

            def _xgroup(g, _):
                i0 = g * 16
                segv = bbuf[pl.ds(i0, 16)]
                for l in range(16):
                    seg = segv[l]
                    row0 = (i0 + l) * D
                    dst0 = seg * D
                    for k in range(D // 16):
                        plsc.addupdate(acc_x.at[pl.ds(dst0 + k * 16, 16)],
                                       xbuf[pl.ds(row0 + k * 16, 16)])
                return 0
            lax.fori_loop(0, CX // 16, _xgroup, 0)

    pltpu.sync_copy(acc_x, xp_hbm.at[wid])
    pltpu.sync_copy(acc_e, ep_hbm.at[wid])
    pltpu.sync_copy(acc_c, cp_hbm.at[wid])


_sc_pool_inner = functools.partial(
    pl.kernel,
    out_type=(
        jax.ShapeDtypeStruct((NW, G * D), jnp.float32),
        jax.ShapeDtypeStruct((NW, G * DE * 16), jnp.float32),
        jax.ShapeDtypeStruct((NW, G * 16), jnp.float32),
    ),
    mesh=plsc.VectorSubcoreMesh(core_axis_name="c", subcore_axis_name="s"),
    compiler_params=pltpu.CompilerParams(use_tc_tiling_on_sc=False),
    scratch_types=[
        pltpu.VMEM((CX * D,), jnp.float32),
        pltpu.VMEM((CX,), jnp.int32),
        pltpu.VMEM((2, DE, EC), jnp.float32),
        pltpu.VMEM((2, EC), jnp.int32),
        pltpu.VMEM((G * D,), jnp.float32),
        pltpu.VMEM((G * DE * 16,), jnp.float32),
        pltpu.VMEM((G * 16,), jnp.float32),
        pltpu.SemaphoreType.DMA,
        pltpu.SemaphoreType.DMA,
    ],
)(_sc_pool_body)


def _sc_pool(x, batch, edge_attr, edge_batch):
    xp, ep, cp = _sc_pool_inner(x.reshape(N * D), batch,
                                edge_attr.T, edge_batch)
    return (xp.reshape(NW, G, D), ep.reshape(NW, G, DE, 16),
            cp.reshape(NW, G, 16))


def _mlp_body(xp_ref, ep_ref, cp_ref, w1a_ref, w1b_ref, w1c_ref,
              b1_ref, w2_ref, b2_ref, o_ref):
    xs = jnp.sum(xp_ref[...], axis=0)
    es = jnp.sum(ep_ref[...], axis=(0, 3))
    cnt = jnp.maximum(jnp.sum(cp_ref[...], axis=(0, 2), keepdims=False),
                      1.0)[:, None]
    h = (jnp.dot(xs * 0.1, w1a_ref[...], preferred_element_type=jnp.float32)
         + jnp.dot(es * 0.05, w1b_ref[...], preferred_element_type=jnp.float32)
         + jnp.dot(es / cnt, w1c_ref[...], preferred_element_type=jnp.float32)
         + b1_ref[...])
    h = jnp.where(h > 0, h, 0.05 * h)
    o = jnp.dot(h, w2_ref[...], preferred_element_type=jnp.float32)
    o_ref[...] = (o + b2_ref[0, 0]) * 0.25


def _mlp(xp, ep, cp, w1a, w1b, w1c, b1, w2p, b2):
    return pl.pallas_call(
        _mlp_body,
        out_shape=jax.ShapeDtypeStruct((G, 128), jnp.float32),
    )(xp, ep, cp, w1a, w1b, w1c, b1, w2p, b2)


def kernel(x, edge_index, edge_attr, batch, edge_batch, W1, b1, W2, b2):
    del edge_index
    batch = batch.astype(jnp.int32)
    edge_batch = edge_batch.astype(jnp.int32)
    xp, ep, cp = _sc_pool(x, batch, edge_attr, edge_batch)
    w1a = W1[:D]
    w1b = W1[D:D + DE]
    w1c = W1[D + DE:]
    w2p = jnp.pad(W2, ((0, 0), (0, 127)))
    o = _mlp(xp, ep, cp, w1a, w1b, w1c,
             b1.reshape(1, H), w2p, b2.reshape(1, 1))
    return o[:, 0:1]


# speedup vs baseline: 12.2056x; 1.4764x over previous
import functools

import jax
import jax.numpy as jnp
from jax import lax
from jax.experimental import pallas as pl
from jax.experimental.pallas import tpu as pltpu
from jax.experimental.pallas import tpu_sc as plsc

N = 10000
E = 320000
D = 128
DE = 16
G = 128
H = 256

_INFO = plsc.get_sparse_core_info()
NC = _INFO.num_cores
NS = _INFO.num_subcores
NW = NC * NS

CX = 80
NCX = N // CX
EC = 1280
NEC = E // EC
NEJ = (NEC + NW - 1) // NW


def _sc_pool_body(x_hbm, b_hbm, ea_hbm, eb_hbm,
                  xp_hbm, ep_hbm, cp_hbm,
                  xbuf, bbuf, abuf, ebbuf, acc_x, acc_e, acc_c,
                  sem0, sem1):
    wid = lax.axis_index("s") * NC + lax.axis_index("c")

    zero16 = jnp.zeros((16,), jnp.float32)
    ones16 = jnp.full((16,), 1.0, jnp.float32)
    sems = [sem0, sem1]

    def _zx(i, _):
        acc_x[pl.ds(i * 16, 16)] = zero16
        return 0
    lax.fori_loop(0, G * D // 16, _zx, 0)

    def _ze(i, _):
        acc_e[pl.ds(i * 16, 16)] = zero16
        return 0
    lax.fori_loop(0, G * DE * 16 // 16, _ze, 0)

    def _zc(i, _):
        acc_c[pl.ds(i * 16, 16)] = zero16
        return 0
    lax.fori_loop(0, G * 16 // 16, _zc, 0)

    def _estart(j, slot):
        c = wid + j * NW
        base = c * EC
        h1 = pltpu.make_async_copy(ea_hbm.at[:, pl.ds(base, EC)],
                                   abuf.at[slot], sems[slot])
        h2 = pltpu.make_async_copy(eb_hbm.at[pl.ds(base, EC)],
                                   ebbuf.at[slot], sems[slot])
        h1.start()
        h2.start()
        return h1, h2

    def _eprocess(slot):
        ab = abuf.at[slot]
        eb = ebbuf.at[slot]
        segv0 = eb[pl.ds(0, 16)]
        init = (segv0[0], jnp.float32(0.0)) + (zero16,) * 16

        def _group(g, carry):
            cur = carry[0]
            cnt = carry[1]
            accs = carry[2:]
            e0 = g * 16
            segv = eb[pl.ds(e0, 16)]
            s0 = segv[0]
            s15 = segv[15]
            fast = jnp.logical_and(s0 == cur, s15 == cur)

            rows = [ab[f, pl.ds(e0, 16)] for f in range(DE)]

            @pl.when(jnp.logical_not(fast))
            def _slow():
                for f in range(DE):
                    plsc.addupdate(acc_e.at[pl.ds((cur * DE + f) * 16, 16)],
                                   accs[f])
                plsc.addupdate(acc_c.at[pl.ds(cur * 16, 16)],
                               lax.broadcast_in_dim(cnt * 0.0625, (16,), ()))
                def _piece(l):
                    sl = segv[l]
                    slv = lax.broadcast_in_dim(sl, (16,), ())
                    mk = jnp.where(segv == slv, ones16, zero16)
                    for f in range(DE):
                        plsc.addupdate(
                            acc_e.at[pl.ds((sl * DE + f) * 16, 16)],
                            rows[f] * mk)
                    plsc.addupdate(acc_c.at[pl.ds(sl * 16, 16)], mk)
                _piece(0)
                for l in range(1, 16):
                    @pl.when(segv[l] != segv[l - 1])
                    def _():
                        _piece(l)

            m = jnp.where(fast, 1.0, 0.0)
            mv = lax.broadcast_in_dim(m, (16,), ())
            new_accs = tuple((accs[f] + rows[f]) * mv for f in range(DE))
            return (jnp.where(fast, cur, s15),
                    jnp.where(fast, cnt + 16.0, 0.0)) + new_accs

        out = lax.fori_loop(0, EC // 16, _group, init)
        cur = out[0]
        cnt = out[1]
        accs = out[2:]
        for f in range(DE):
            plsc.addupdate(acc_e.at[pl.ds((cur * DE + f) * 16, 16)], accs[f])
        plsc.addupdate(acc_c.at[pl.ds(cur * 16, 16)],
                       lax.broadcast_in_dim(cnt * 0.0625, (16,), ()))

    def _ewait(slot):
        pltpu.make_async_copy(ea_hbm.at[:, pl.ds(0, EC)], abuf.at[slot],
                              sems[slot]).wait()
        pltpu.make_async_copy(eb_hbm.at[pl.ds(0, EC)], ebbuf.at[slot],
                              sems[slot]).wait()

    _estart(0, 0)
    for j in range(NEJ):
        if j + 1 < NEJ:
            @pl.when(wid + (j + 1) * NW < NEC)
            def _():
                _estart(j + 1, (j + 1) % 2)

        @pl.when(wid + j * NW < NEC)
        def _():
            _ewait(j % 2)
            _eprocess(j % 2)

    for j in range((NCX + NW - 1) // NW):
        c = wid + j * NW

        @pl.when(c < NCX)
        def _():
            base = c * CX
            pltpu.sync_copy(x_hbm.at[pl.ds(base * D, CX * D)], xbuf)
            pltpu.sync_copy(b_hbm.at[pl.ds(base, CX)], bbuf)

            def _xgroup(g, _):
                i0 = g * 16
                segv = bbuf[pl.ds(i0, 16)]
                for l in range(16):
                    seg = segv[l]
                    row0 = (i0 + l) * D
                    dst0 = seg * D
                    for k in range(D // 16):
                        plsc.addupdate(acc_x.at[pl.ds(dst0 + k * 16, 16)],
                                       xbuf[pl.ds(row0 + k * 16, 16)])
                return 0
            lax.fori_loop(0, CX // 16, _xgroup, 0)

    pltpu.sync_copy(acc_x, xp_hbm.at[wid])
    pltpu.sync_copy(acc_e, ep_hbm.at[wid])
    pltpu.sync_copy(acc_c, cp_hbm.at[wid])


_sc_pool_inner = functools.partial(
    pl.kernel,
    out_type=(
        jax.ShapeDtypeStruct((NW, G * D), jnp.float32),
        jax.ShapeDtypeStruct((NW, G * DE * 16), jnp.float32),
        jax.ShapeDtypeStruct((NW, G * 16), jnp.float32),
    ),
    mesh=plsc.VectorSubcoreMesh(core_axis_name="c", subcore_axis_name="s"),
    compiler_params=pltpu.CompilerParams(use_tc_tiling_on_sc=False),
    scratch_types=[
        pltpu.VMEM((CX * D,), jnp.float32),
        pltpu.VMEM((CX,), jnp.int32),
        pltpu.VMEM((2, DE, EC), jnp.float32),
        pltpu.VMEM((2, EC), jnp.int32),
        pltpu.VMEM((G * D,), jnp.float32),
        pltpu.VMEM((G * DE * 16,), jnp.float32),
        pltpu.VMEM((G * 16,), jnp.float32),
        pltpu.SemaphoreType.DMA,
        pltpu.SemaphoreType.DMA,
    ],
)(_sc_pool_body)


def _sc_pool(x, batch, edge_attr, edge_batch):
    return _sc_pool_inner(x.reshape(N * D), batch, edge_attr.T, edge_batch)


def _mlp_body(xp_ref, ep_ref, cp_ref, w1a_ref, w1b_ref, w1c_ref,
              b1_ref, w2_ref, b2_ref, o_ref):
    xs = jnp.sum(xp_ref[...], axis=0).reshape(G, D)
    w = jnp.sum(ep_ref[...], axis=0).reshape(G, DE * 16)
    c = jnp.sum(cp_ref[...], axis=0)
    fold = (lax.broadcasted_iota(jnp.int32, (DE * 16, DE), 0) // 16
            == lax.broadcasted_iota(jnp.int32, (DE * 16, DE), 1)
            ).astype(jnp.float32)
    es = jnp.dot(w, fold, preferred_element_type=jnp.float32)
    sel = (lax.broadcasted_iota(jnp.int32, (G, G * 16), 1) // 16
           == lax.broadcasted_iota(jnp.int32, (G, G * 16), 0)
           ).astype(jnp.float32)
    cnt = jnp.maximum(
        jnp.dot(sel, c, preferred_element_type=jnp.float32), 1.0)[:, None]
    h = (jnp.dot(xs * 0.1, w1a_ref[...], preferred_element_type=jnp.float32)
         + jnp.dot(es * 0.05, w1b_ref[...], preferred_element_type=jnp.float32)
         + jnp.dot(es / cnt, w1c_ref[...], preferred_element_type=jnp.float32)
         + b1_ref[...])
    h = jnp.where(h > 0, h, 0.05 * h)
    o = jnp.dot(h, w2_ref[...], preferred_element_type=jnp.float32)
    o_ref[...] = (o + b2_ref[0, 0]) * 0.25


def _mlp(xp, ep, cp, w1a, w1b, w1c, b1, w2p, b2):
    return pl.pallas_call(
        _mlp_body,
        out_shape=jax.ShapeDtypeStruct((G, 128), jnp.float32),
    )(xp, ep, cp, w1a, w1b, w1c, b1, w2p, b2)


def kernel(x, edge_index, edge_attr, batch, edge_batch, W1, b1, W2, b2):
    del edge_index
    batch = batch.astype(jnp.int32)
    edge_batch = edge_batch.astype(jnp.int32)
    xp, ep, cp = _sc_pool(x, batch, edge_attr, edge_batch)
    w1a = W1[:D]
    w1b = W1[D:D + DE]
    w1c = W1[D + DE:]
    w2p = jnp.pad(W2, ((0, 0), (0, 127)))
    o = _mlp(xp, ep, cp, w1a, w1b, w1c,
             b1.reshape(1, H), w2p, b2.reshape(1, 1))
    return o[:, 0:1]


# speedup vs baseline: 14.7935x; 1.2120x over previous
import functools

import jax
import jax.numpy as jnp
from jax import lax
from jax.experimental import pallas as pl
from jax.experimental.pallas import tpu as pltpu
from jax.experimental.pallas import tpu_sc as plsc

N = 10000
E = 320000
D = 128
DE = 16
G = 128
H = 256

_INFO = plsc.get_sparse_core_info()
NC = _INFO.num_cores
NS = _INFO.num_subcores
NW = NC * NS

CX = 80
NCX = N // CX
EC = 1280
NEC = E // EC
NEJ = (NEC + NW - 1) // NW


def _sc_pool_body(x_hbm, b_hbm, ea_hbm, eb_hbm,
                  xp_hbm, ep_hbm, cp_hbm,
                  xbuf, bbuf, abuf0, abuf1, ebbuf0, ebbuf1,
                  acc_x, acc_e, acc_c,
                  sem0, sem1):
    abufs = [abuf0, abuf1]
    ebbufs = [ebbuf0, ebbuf1]
    wid = lax.axis_index("s") * NC + lax.axis_index("c")

    zero16 = jnp.zeros((16,), jnp.float32)
    ones16 = jnp.full((16,), 1.0, jnp.float32)
    sems = [sem0, sem1]

    def _zx(i, _):
        acc_x[pl.ds(i * 16, 16)] = zero16
        return 0
    lax.fori_loop(0, G * D // 16, _zx, 0)

    def _ze(i, _):
        acc_e[pl.ds(i * 16, 16)] = zero16
        return 0
    lax.fori_loop(0, G * DE * 16 // 16, _ze, 0)

    def _zc(i, _):
        acc_c[pl.ds(i * 16, 16)] = zero16
        return 0
    lax.fori_loop(0, G * 16 // 16, _zc, 0)

    def _estart(j, slot):
        c = wid + j * NW
        base = c * EC
        h1 = pltpu.make_async_copy(ea_hbm.at[:, pl.ds(base, EC)],
                                   abufs[slot], sems[slot])
        h2 = pltpu.make_async_copy(eb_hbm.at[pl.ds(base, EC)],
                                   ebbufs[slot], sems[slot])
        h1.start()
        h2.start()
        return h1, h2

    def _eprocess(slot):
        ab = abufs[slot]
        eb = ebbufs[slot]
        segv0 = eb[pl.ds(0, 16)]
        init = (segv0[0], jnp.float32(0.0)) + (zero16,) * 16

        def _group(g, carry):
            cur = carry[0]
            cnt = carry[1]
            accs = carry[2:]
            e0 = g * 16
            segv = eb[pl.ds(e0, 16)]
            s0 = segv[0]
            s15 = segv[15]
            fast = jnp.logical_and(s0 == cur, s15 == cur)

            rows = [ab[f, pl.ds(e0, 16)] for f in range(DE)]

            @pl.when(jnp.logical_not(fast))
            def _slow():
                for f in range(DE):
                    plsc.addupdate(acc_e.at[pl.ds((cur * DE + f) * 16, 16)],
                                   accs[f])
                plsc.addupdate(acc_c.at[pl.ds(cur * 16, 16)],
                               lax.broadcast_in_dim(cnt * 0.0625, (16,), ()))
                def _piece(l):
                    sl = segv[l]
                    slv = lax.broadcast_in_dim(sl, (16,), ())
                    mk = jnp.where(segv == slv, ones16, zero16)
                    for f in range(DE):
                        plsc.addupdate(
                            acc_e.at[pl.ds((sl * DE + f) * 16, 16)],
                            rows[f] * mk)
                    plsc.addupdate(acc_c.at[pl.ds(sl * 16, 16)], mk)
                _piece(0)
                for l in range(1, 16):
                    @pl.when(segv[l] != segv[l - 1])
                    def _():
                        _piece(l)

            m = jnp.where(fast, 1.0, 0.0)
            mv = lax.broadcast_in_dim(m, (16,), ())
            new_accs = tuple((accs[f] + rows[f]) * mv for f in range(DE))
            return (jnp.where(fast, cur, s15),
                    jnp.where(fast, cnt + 16.0, 0.0)) + new_accs

        out = lax.fori_loop(0, EC // 16, _group, init)
        cur = out[0]
        cnt = out[1]
        accs = out[2:]
        for f in range(DE):
            plsc.addupdate(acc_e.at[pl.ds((cur * DE + f) * 16, 16)], accs[f])
        plsc.addupdate(acc_c.at[pl.ds(cur * 16, 16)],
                       lax.broadcast_in_dim(cnt * 0.0625, (16,), ()))

    def _ewait(slot):
        pltpu.make_async_copy(ea_hbm.at[:, pl.ds(0, EC)], abufs[slot],
                              sems[slot]).wait()
        pltpu.make_async_copy(eb_hbm.at[pl.ds(0, EC)], ebbufs[slot],
                              sems[slot]).wait()

    _estart(0, 0)
    for j in range(NEJ):
        if j + 1 < NEJ:
            @pl.when(wid + (j + 1) * NW < NEC)
            def _():
                _estart(j + 1, (j + 1) % 2)

        @pl.when(wid + j * NW < NEC)
        def _():
            _ewait(j % 2)
            _eprocess(j % 2)

    for j in range((NCX + NW - 1) // NW):
        c = wid + j * NW

        @pl.when(c < NCX)
        def _():
            base = c * CX
            pltpu.sync_copy(x_hbm.at[pl.ds(base * D, CX * D)], xbuf)
            pltpu.sync_copy(b_hbm.at[pl.ds(base, CX)], bbuf)

            def _xgroup(g, _):
                i0 = g * 16
                segv = bbuf[pl.ds(i0, 16)]
                for l in range(16):
                    seg = segv[l]
                    row0 = (i0 + l) * D
                    dst0 = seg * D
                    for k in range(D // 16):
                        plsc.addupdate(acc_x.at[pl.ds(dst0 + k * 16, 16)],
                                       xbuf[pl.ds(row0 + k * 16, 16)])
                return 0
            lax.fori_loop(0, CX // 16, _xgroup, 0)

    pltpu.sync_copy(acc_x, xp_hbm.at[wid])
    pltpu.sync_copy(acc_e, ep_hbm.at[wid])
    pltpu.sync_copy(acc_c, cp_hbm.at[wid])


_sc_pool_inner = functools.partial(
    pl.kernel,
    out_type=(
        jax.ShapeDtypeStruct((NW, G * D), jnp.float32),
        jax.ShapeDtypeStruct((NW, G * DE * 16), jnp.float32),
        jax.ShapeDtypeStruct((NW, G * 16), jnp.float32),
    ),
    mesh=plsc.VectorSubcoreMesh(core_axis_name="c", subcore_axis_name="s"),
    compiler_params=pltpu.CompilerParams(use_tc_tiling_on_sc=True),
    scratch_types=[
        pltpu.VMEM((CX * D,), jnp.float32),
        pltpu.VMEM((CX,), jnp.int32),
        pltpu.VMEM((DE, EC), jnp.float32),
        pltpu.VMEM((DE, EC), jnp.float32),
        pltpu.VMEM((EC,), jnp.int32),
        pltpu.VMEM((EC,), jnp.int32),
        pltpu.VMEM((G * D,), jnp.float32),
        pltpu.VMEM((G * DE * 16,), jnp.float32),
        pltpu.VMEM((G * 16,), jnp.float32),
        pltpu.SemaphoreType.DMA,
        pltpu.SemaphoreType.DMA,
    ],
)(_sc_pool_body)


def _sc_pool(x, batch, edge_attr, edge_batch):
    return _sc_pool_inner(x.reshape(N * D), batch, edge_attr.T, edge_batch)


def _mlp_body(xp_ref, ep_ref, cp_ref, w1a_ref, w1b_ref, w1c_ref,
              b1_ref, w2_ref, b2_ref, o_ref):
    xs = jnp.sum(xp_ref[...], axis=0).reshape(G, D)
    w = jnp.sum(ep_ref[...], axis=0).reshape(G, DE * 16)
    c = jnp.sum(cp_ref[...], axis=0)
    fold = (lax.broadcasted_iota(jnp.int32, (DE * 16, DE), 0) // 16
            == lax.broadcasted_iota(jnp.int32, (DE * 16, DE), 1)
            ).astype(jnp.float32)
    es = jnp.dot(w, fold, preferred_element_type=jnp.float32)
    sel = (lax.broadcasted_iota(jnp.int32, (G, G * 16), 1) // 16
           == lax.broadcasted_iota(jnp.int32, (G, G * 16), 0)
           ).astype(jnp.float32)
    cnt = jnp.maximum(
        jnp.dot(sel, c, preferred_element_type=jnp.float32), 1.0)[:, None]
    h = (jnp.dot(xs * 0.1, w1a_ref[...], preferred_element_type=jnp.float32)
         + jnp.dot(es * 0.05, w1b_ref[...], preferred_element_type=jnp.float32)
         + jnp.dot(es / cnt, w1c_ref[...], preferred_element_type=jnp.float32)
         + b1_ref[...])
    h = jnp.where(h > 0, h, 0.05 * h)
    o = jnp.dot(h, w2_ref[...], preferred_element_type=jnp.float32)
    o_ref[...] = (o + b2_ref[0, 0]) * 0.25


def _mlp(xp, ep, cp, w1a, w1b, w1c, b1, w2p, b2):
    return pl.pallas_call(
        _mlp_body,
        out_shape=jax.ShapeDtypeStruct((G, 128), jnp.float32),
    )(xp, ep, cp, w1a, w1b, w1c, b1, w2p, b2)


def kernel(x, edge_index, edge_attr, batch, edge_batch, W1, b1, W2, b2):
    del edge_index
    batch = batch.astype(jnp.int32)
    edge_batch = edge_batch.astype(jnp.int32)
    xp, ep, cp = _sc_pool(x, batch, edge_attr, edge_batch)
    w1a = W1[:D]
    w1b = W1[D:D + DE]
    w1c = W1[D + DE:]
    w2p = jnp.pad(W2, ((0, 0), (0, 127)))
    o = _mlp(xp, ep, cp, w1a, w1b, w1c,
             b1.reshape(1, H), w2p, b2.reshape(1, 1))
    return o[:, 0:1]
